# baseline (speedup 1.0000x reference)
import functools

import jax
import jax.numpy as jnp
from jax import lax
from jax.experimental import pallas as pl
from jax.experimental.pallas import tpu as pltpu
from jax.experimental.pallas import tpu_sc as plsc

NUM_SCENES = 1000
D = 128
BATCH = 16384

_INFO = plsc.get_sparse_core_info()
_NC = _INFO.num_cores
_NS = _INFO.num_subcores
_NW = _NC * _NS
_B_C0 = 464
_B_C1 = (BATCH // _NS) - _B_C0
_B_MAX = max(_B_C0, _B_C1)


def _make_gather():
    mesh = plsc.VectorSubcoreMesh(core_axis_name="c", subcore_axis_name="s")

    @functools.partial(
        pl.kernel,
        mesh=mesh,
        out_type=jax.ShapeDtypeStruct((BATCH, D), jnp.float32),
        scratch_types=[
            pltpu.VMEM((_B_MAX,), jnp.int32),
            pltpu.VMEM((_B_MAX, D), jnp.float32),
            pltpu.SemaphoreType.DMA,
        ],
    )
    def gather_kernel(idx_hbm, table_hbm, out_hbm, idx_v, rows_v, sem):
        c = lax.axis_index("c")
        s = lax.axis_index("s")

        def work(nb, base):
            pltpu.sync_copy(idx_hbm.at[pl.ds(base, nb)], idx_v.at[pl.ds(0, nb)])
            pltpu.async_copy(table_hbm.at[idx_v.at[pl.ds(0, nb)]],
                             rows_v.at[pl.ds(0, nb)], sem).wait()
            pltpu.sync_copy(rows_v.at[pl.ds(0, nb)],
                            out_hbm.at[pl.ds(base, nb)])

        @pl.when(c == 0)
        def _():
            work(_B_C0, s * (_B_C0 + _B_C1))

        @pl.when(c != 0)
        def _():
            work(_B_C1, s * (_B_C0 + _B_C1) + _B_C0)

    return gather_kernel


_gather = _make_gather()


def kernel(scene_id, embedding_weight):
    if scene_id.ndim > 1:
        scene_id = jnp.squeeze(scene_id, axis=-1)
    return _gather(scene_id.astype(jnp.int32), embedding_weight)
